# baseline (speedup 1.0000x reference)
import functools

import jax
import jax.numpy as jnp
from jax import lax
from jax.experimental import pallas as pl
from jax.experimental.pallas import tpu as pltpu
from jax.experimental.pallas import tpu_sc as plsc

B = 128
S = 512
H = 768
NC = 2
NS = 16
NW = NC * NS
PW = S // NW
G = 2
GPW = G * PW
NG = B // G
NV = H // 16


@functools.partial(
    pl.kernel,
    mesh=plsc.VectorSubcoreMesh(core_axis_name="c", subcore_axis_name="s"),
    out_type=jax.ShapeDtypeStruct((B, S, H), jnp.float32),
    compiler_params=pltpu.CompilerParams(
        use_tc_tiling_on_sc=True, needs_layout_passes=False),
    scratch_types=[
        pltpu.VMEM((NG * GPW,), jnp.int32),
        pltpu.VMEM((NG * GPW,), jnp.int32),
        pltpu.VMEM((PW, H), jnp.float32),
        pltpu.VMEM((2, H), jnp.float32),
    ] + [pltpu.VMEM((GPW, H), jnp.float32)] * 4
      + [pltpu.SemaphoreType.DMA] * 8,
)
def _emb(xr_hbm, tok_hbm, pos_hbm, typ_hbm, out_hbm,
         idx_all, tt_all, pos_v, typ_v, *ring):
    toks = ring[:4]
    gss = ring[4:8]
    oss = ring[8:12]
    wid = lax.axis_index("s") * NC + lax.axis_index("c")
    pw = wid * PW

    NT = NW * NG * GPW
    pltpu.sync_copy(xr_hbm.at[pl.ds(wid * NG * GPW, NG * GPW)], idx_all)
    pltpu.sync_copy(xr_hbm.at[pl.ds(NT + wid * NG * GPW, NG * GPW)], tt_all)
    pltpu.sync_copy(pos_hbm.at[pl.ds(pw, PW)], pos_v)
    pltpu.sync_copy(typ_hbm, typ_v)

    def start_gather(g, buf, sem):
        pltpu.async_copy(tok_hbm.at[idx_all.at[pl.ds(g * GPW, GPW)]], buf, sem)

    def wait_gather(g, buf, sem):
        pltpu.make_async_copy(tok_hbm.at[idx_all.at[pl.ds(g * GPW, GPW)]], buf, sem).wait()

    def start_writeout(g, buf, sem):
        for j in range(G):
            pltpu.async_copy(
                buf.at[pl.ds(j * PW, PW)],
                out_hbm.at[g * G + j, pl.ds(pw, PW)], sem)

    def wait_writeout(g, buf, sem):
        for j in range(G):
            pltpu.make_async_copy(
                buf.at[pl.ds(j * PW, PW)],
                out_hbm.at[g * G + j, pl.ds(pw, PW)], sem).wait()

    def compute(g, buf):
        for j in range(G):
            row0 = g * GPW + j * PW

            @plsc.parallel_loop(0, H, step=128)
            def chunk_body(c0):
                t0 = [typ_v[0, pl.ds(c0 + v * 16, 16)] for v in range(8)]
                dt = [typ_v[1, pl.ds(c0 + v * 16, 16)] - t0[v]
                      for v in range(8)]

                @plsc.parallel_loop(0, PW)
                def row_body(rp):
                    ttb = plsc.load_gather(
                        tt_all, [jnp.full((16,), row0 + rp, jnp.int32)])
                    ttf = ttb.astype(jnp.float32)
                    tmp = [t0[v] + ttf * dt[v] for v in range(8)]
                    for v in range(8):
                        sl = pl.ds(c0 + v * 16, 16)
                        vec = pos_v[rp, pl.ds(c0 + v * 16, 16)] + tmp[v]
                        plsc.addupdate(buf.at[j * PW + rp, sl], vec)

    NBUF = 4
    DEPTH = 2

    for k in range(DEPTH):
        start_gather(k, toks[k], gss[k])

    def group_body(gg, carry):
        for i in range(NBUF):
            g = gg * NBUF + i
            gp = g + DEPTH
            pp = (i + DEPTH) % NBUF

            @pl.when(gp < NG)
            def _():
                @pl.when(gp >= NBUF)
                def _():
                    wait_writeout(gp - NBUF, toks[pp], oss[pp])
                start_gather(gp, toks[pp], gss[pp])

            wait_gather(g, toks[i], gss[i])
            compute(g, toks[i])
            start_writeout(g, toks[i], oss[i])
        return carry

    lax.fori_loop(0, NG // NBUF, group_body, 0)

    for k in range(NBUF):
        wait_writeout(NG - NBUF + k, toks[k], oss[k])


def kernel(x, token_type_ids, token_table, pos_table, type_table):
    xtt = (jnp.stack([x, token_type_ids])
             .reshape(2, NG, G, NW, PW)
             .transpose(0, 3, 1, 2, 4)
             .reshape(2 * NW * NG * GPW))
    return _emb(xtt, token_table, pos_table[:S], type_table)


# speedup vs baseline: 1.0765x; 1.0765x over previous
import functools

import jax
import jax.numpy as jnp
from jax import lax
from jax.experimental import pallas as pl
from jax.experimental.pallas import tpu as pltpu
from jax.experimental.pallas import tpu_sc as plsc

B = 128
S = 512
H = 768
NC = 2
NS = 16
NW = NC * NS
PW = S // NW
G = 1
GPW = G * PW
NG = B // G
NV = H // 16


@functools.partial(
    pl.kernel,
    mesh=plsc.VectorSubcoreMesh(core_axis_name="c", subcore_axis_name="s"),
    out_type=jax.ShapeDtypeStruct((B, S, H), jnp.float32),
    compiler_params=pltpu.CompilerParams(
        use_tc_tiling_on_sc=True, needs_layout_passes=False),
    scratch_types=[
        pltpu.VMEM((NG * GPW,), jnp.int32),
        pltpu.VMEM((NG * GPW,), jnp.int32),
        pltpu.VMEM((PW, H), jnp.float32),
        pltpu.VMEM((2, H), jnp.float32),
    ] + [pltpu.VMEM((GPW, H), jnp.float32)] * 8
      + [pltpu.SemaphoreType.DMA] * 16,
)
def _emb(xr_hbm, tok_hbm, pos_hbm, typ_hbm, out_hbm,
         idx_all, tt_all, pos_v, typ_v, *ring):
    toks = ring[:8]
    gss = ring[8:16]
    oss = ring[16:24]
    wid = lax.axis_index("s") * NC + lax.axis_index("c")
    pw = wid * PW

    NT = NW * NG * GPW
    pltpu.sync_copy(xr_hbm.at[pl.ds(wid * NG * GPW, NG * GPW)], idx_all)
    pltpu.sync_copy(xr_hbm.at[pl.ds(NT + wid * NG * GPW, NG * GPW)], tt_all)
    pltpu.sync_copy(pos_hbm.at[pl.ds(pw, PW)], pos_v)
    pltpu.sync_copy(typ_hbm, typ_v)

    def start_gather(g, buf, sem):
        pltpu.async_copy(tok_hbm.at[idx_all.at[pl.ds(g * GPW, GPW)]], buf, sem)

    def wait_gather(g, buf, sem):
        pltpu.make_async_copy(tok_hbm.at[idx_all.at[pl.ds(g * GPW, GPW)]], buf, sem).wait()

    def start_writeout(g, buf, sem):
        for j in range(G):
            pltpu.async_copy(
                buf.at[pl.ds(j * PW, PW)],
                out_hbm.at[g * G + j, pl.ds(pw, PW)], sem)

    def wait_writeout(g, buf, sem):
        for j in range(G):
            pltpu.make_async_copy(
                buf.at[pl.ds(j * PW, PW)],
                out_hbm.at[g * G + j, pl.ds(pw, PW)], sem).wait()

    def compute(g, buf):
        for j in range(G):
            row0 = g * GPW + j * PW

            @plsc.parallel_loop(0, H, step=128)
            def chunk_body(c0):
                t0 = [typ_v[0, pl.ds(c0 + v * 16, 16)] for v in range(8)]
                dt = [typ_v[1, pl.ds(c0 + v * 16, 16)] - t0[v]
                      for v in range(8)]

                @plsc.parallel_loop(0, PW)
                def row_body(rp):
                    ttb = plsc.load_gather(
                        tt_all, [jnp.full((16,), row0 + rp, jnp.int32)])
                    ttf = ttb.astype(jnp.float32)
                    tmp = [t0[v] + ttf * dt[v] for v in range(8)]
                    for v in range(8):
                        sl = pl.ds(c0 + v * 16, 16)
                        vec = pos_v[rp, pl.ds(c0 + v * 16, 16)] + tmp[v]
                        plsc.addupdate(buf.at[j * PW + rp, sl], vec)

    NBUF = 8
    DEPTH = 5

    for k in range(DEPTH):
        start_gather(k, toks[k], gss[k])

    def group_body(gg, carry):
        for i in range(NBUF):
            g = gg * NBUF + i
            gp = g + DEPTH
            pp = (i + DEPTH) % NBUF

            @pl.when(gp < NG)
            def _():
                @pl.when(gp >= NBUF)
                def _():
                    wait_writeout(gp - NBUF, toks[pp], oss[pp])
                start_gather(gp, toks[pp], gss[pp])

            wait_gather(g, toks[i], gss[i])
            compute(g, toks[i])
            start_writeout(g, toks[i], oss[i])
        return carry

    lax.fori_loop(0, NG // NBUF, group_body, 0)

    for k in range(NBUF):
        wait_writeout(NG - NBUF + k, toks[k], oss[k])


def kernel(x, token_type_ids, token_table, pos_table, type_table):
    xtt = (jnp.stack([x, token_type_ids])
             .reshape(2, NG, G, NW, PW)
             .transpose(0, 3, 1, 2, 4)
             .reshape(2 * NW * NG * GPW))
    return _emb(xtt, token_table, pos_table[:S], type_table)
